# Initial kernel scaffold; baseline (speedup 1.0000x reference)
import functools
import jax
import jax.numpy as jnp
from jax import lax
from jax.experimental import pallas as pl
from jax.experimental.pallas import tpu as pltpu
from jax.experimental.pallas import tpu_sc as plsc

VOCAB = 1000000
DIM = 64
BATCH = 16384
NNEG = 20

NC = 2
NS = 16
L = 16
NW = NC * NS
BPW = BATCH // NW
CHUNK = 32
NCHUNK = BPW // CHUNK
NEG_IDX_ROWS = CHUNK * NNEG // 128


def _sc_kernel_body(tgt_hbm, ctx_hbm, neg_hbm, wt_hbm, wc_hbm,
                    pos_hbm, neg_out_hbm,
                    tgt_idx_v, ctx_idx_v, neg_idx_v,
                    trow_v, crow_v, negrows_v,
                    pos_out_v, negsum_out_v, sem):
    wid = lax.axis_index("s") * NC + lax.axis_index("c")
    wbase = wid * BPW

    def chunk_body(ch, _):
        base = wbase + ch * CHUNK
        pltpu.sync_copy(tgt_hbm.at[pl.ds(base, CHUNK)], tgt_idx_v)
        pltpu.sync_copy(ctx_hbm.at[pl.ds(base, CHUNK)], ctx_idx_v)
        nrow0 = (wid * NCHUNK + ch) * NEG_IDX_ROWS
        pltpu.sync_copy(neg_hbm.at[pl.ds(nrow0, NEG_IDX_ROWS)], neg_idx_v)
        cps = [
            pltpu.async_copy(wt_hbm.at[tgt_idx_v], trow_v, sem),
            pltpu.async_copy(wc_hbm.at[ctx_idx_v], crow_v, sem),
        ]
        for j in range(NEG_IDX_ROWS):
            cps.append(
                pltpu.async_copy(wc_hbm.at[neg_idx_v.at[j]],
                                 negrows_v.at[pl.ds(j * 128, 128)], sem))
        for cp in cps:
            cp.wait()

        lane = lax.iota(jnp.int32, L)
        for g in range(CHUNK // L):
            def row_body(r16, carry):
                pos_vec, neg_vec = carry
                r = g * L + r16
                t = [trow_v[r, pl.ds(k * L, L)] for k in range(DIM // L)]
                c = [crow_v[r, pl.ds(k * L, L)] for k in range(DIM // L)]
                pv = t[0] * c[0]
                for k in range(1, DIM // L):
                    pv = pv + t[k] * c[k]
                nbase = r * NNEG
                acc = [negrows_v[nbase, pl.ds(k * L, L)]
                       for k in range(DIM // L)]
                for n in range(1, NNEG):
                    for k in range(DIM // L):
                        acc[k] = acc[k] + negrows_v[nbase + n, pl.ds(k * L, L)]
                nv = acc[0] * t[0]
                for k in range(1, DIM // L):
                    nv = nv + acc[k] * t[k]
                ps = lax.reduce(pv, jnp.float32(0), lax.add, (0,))
                ns = lax.reduce(nv, jnp.float32(0), lax.add, (0,))
                sel = lane == r16
                pos_vec = jnp.where(sel, jnp.full((L,), ps, jnp.float32),
                                    pos_vec)
                neg_vec = jnp.where(sel, jnp.full((L,), ns, jnp.float32),
                                    neg_vec)
                return pos_vec, neg_vec

            zeros = jnp.zeros((L,), jnp.float32)
            pos_vec, neg_vec = lax.fori_loop(0, L, row_body, (zeros, zeros))
            off = ch * CHUNK + g * L
            pos_out_v[pl.ds(off, L)] = pos_vec
            negsum_out_v[pl.ds(off, L)] = neg_vec
        return 0

    lax.fori_loop(0, NCHUNK, chunk_body, 0)
    pltpu.sync_copy(pos_out_v, pos_hbm.at[pl.ds(wbase, BPW)])
    pltpu.sync_copy(negsum_out_v, neg_out_hbm.at[pl.ds(wbase, BPW)])


def _make_sc_call():
    mesh = plsc.VectorSubcoreMesh(core_axis_name="c", subcore_axis_name="s",
                                  num_cores=NC, num_subcores=NS)
    return pl.kernel(
        _sc_kernel_body,
        out_type=(
            jax.ShapeDtypeStruct((BATCH,), jnp.float32),
            jax.ShapeDtypeStruct((BATCH,), jnp.float32),
        ),
        mesh=mesh,
        scratch_types=[
            pltpu.VMEM((CHUNK,), jnp.int32),
            pltpu.VMEM((CHUNK,), jnp.int32),
            pltpu.VMEM((NEG_IDX_ROWS, 128), jnp.int32),
            pltpu.VMEM((CHUNK, DIM), jnp.float32),
            pltpu.VMEM((CHUNK, DIM), jnp.float32),
            pltpu.VMEM((CHUNK * NNEG, DIM), jnp.float32),
            pltpu.VMEM((BPW,), jnp.float32),
            pltpu.VMEM((BPW,), jnp.float32),
            pltpu.SemaphoreType.DMA,
        ],
    )


def _loss_kernel(pos_ref, neg_ref, out_ref):
    p = pos_ref[...]
    n = -neg_ref[...]
    lsp = jnp.minimum(p, 0.0) - jnp.log1p(jnp.exp(-jnp.abs(p)))
    lsn = jnp.minimum(n, 0.0) - jnp.log1p(jnp.exp(-jnp.abs(n)))
    out_ref[0, 0] = -(jnp.sum(lsp) + jnp.sum(lsn))


@jax.jit
def kernel(target_word, context_word, negative_example, W_target, W_context):
    neg_flat = negative_example.reshape(BATCH * NNEG // 128, 128)
    sc = _make_sc_call()
    pos, negsum = sc(target_word.astype(jnp.int32),
                     context_word.astype(jnp.int32),
                     neg_flat.astype(jnp.int32),
                     W_target, W_context)
    loss = pl.pallas_call(
        _loss_kernel,
        out_shape=jax.ShapeDtypeStruct((1, 1), jnp.float32),
    )(pos.reshape(128, 128), negsum.reshape(128, 128))
    return loss[0, 0]


# speedup vs baseline: 5.0778x; 5.0778x over previous
import jax
import jax.numpy as jnp
from jax import lax
from jax.experimental import pallas as pl
from jax.experimental.pallas import tpu as pltpu
from jax.experimental.pallas import tpu_sc as plsc

VOCAB = 1000000
DIM = 64
BATCH = 16384
NNEG = 20

NC = 2
NS = 16
L = 16
NW = NC * NS
BPW = BATCH // NW
CHUNK = 64
NCHUNK = BPW // CHUNK
NIDX = CHUNK * NNEG


def _sc_kernel_body(tgt_hbm, ctx_hbm, neg_hbm, wt_hbm, wc_hbm,
                    embt_hbm, embc_hbm, negacc_hbm,
                    tgt_idx_v, ctx_idx_v, neg_idx_v,
                    trow_v, crow_v, negrows_v, negacc_v, sem):
    wid = lax.axis_index("s") * NC + lax.axis_index("c")
    wbase = wid * BPW

    def chunk_body(ch, _):
        base = wbase + ch * CHUNK
        pltpu.sync_copy(tgt_hbm.at[pl.ds(base, CHUNK)], tgt_idx_v)
        pltpu.sync_copy(ctx_hbm.at[pl.ds(base, CHUNK)], ctx_idx_v)
        pltpu.sync_copy(neg_hbm.at[pl.ds(base * NNEG, NIDX)], neg_idx_v)
        cps = [
            pltpu.async_copy(wt_hbm.at[tgt_idx_v], trow_v, sem),
            pltpu.async_copy(wc_hbm.at[ctx_idx_v], crow_v, sem),
        ]
        for j in range(NIDX // 128):
            cps.append(
                pltpu.async_copy(wc_hbm.at[neg_idx_v.at[pl.ds(j * 128, 128)]],
                                 negrows_v.at[pl.ds(j * 128, 128)], sem))
        for cp in cps:
            cp.wait()

        def row_body(r, _):
            nbase = r * NNEG
            for k in range(DIM // L):
                acc = negrows_v[nbase, pl.ds(k * L, L)]
                for n in range(1, NNEG):
                    acc = acc + negrows_v[nbase + n, pl.ds(k * L, L)]
                negacc_v[r, pl.ds(k * L, L)] = acc
            return 0

        lax.fori_loop(0, CHUNK, row_body, 0)

        pltpu.sync_copy(trow_v, embt_hbm.at[pl.ds(base, CHUNK)])
        pltpu.sync_copy(crow_v, embc_hbm.at[pl.ds(base, CHUNK)])
        pltpu.sync_copy(negacc_v, negacc_hbm.at[pl.ds(base, CHUNK)])
        return 0

    lax.fori_loop(0, NCHUNK, chunk_body, 0)


def _make_sc_call():
    mesh = plsc.VectorSubcoreMesh(core_axis_name="c", subcore_axis_name="s",
                                  num_cores=NC, num_subcores=NS)
    return pl.kernel(
        _sc_kernel_body,
        out_type=(
            jax.ShapeDtypeStruct((BATCH, DIM), jnp.float32),
            jax.ShapeDtypeStruct((BATCH, DIM), jnp.float32),
            jax.ShapeDtypeStruct((BATCH, DIM), jnp.float32),
        ),
        mesh=mesh,
        compiler_params=pltpu.CompilerParams(use_tc_tiling_on_sc=False),
        scratch_types=[
            pltpu.VMEM((CHUNK,), jnp.int32),
            pltpu.VMEM((CHUNK,), jnp.int32),
            pltpu.VMEM((NIDX,), jnp.int32),
            pltpu.VMEM((CHUNK, DIM), jnp.float32),
            pltpu.VMEM((CHUNK, DIM), jnp.float32),
            pltpu.VMEM((NIDX, DIM), jnp.float32),
            pltpu.VMEM((CHUNK, DIM), jnp.float32),
            pltpu.SemaphoreType.DMA,
        ],
    )


ROWS_PER_STEP = 2048


def _loss_kernel(t_ref, c_ref, n_ref, out_ref):
    i = pl.program_id(0)

    @pl.when(i == 0)
    def _():
        out_ref[...] = jnp.zeros_like(out_ref)

    t = t_ref[...]
    p = jnp.sum(t * c_ref[...], axis=1)
    q = -jnp.sum(t * n_ref[...], axis=1)
    lsp = jnp.minimum(p, 0.0) - jnp.log1p(jnp.exp(-jnp.abs(p)))
    lsq = jnp.minimum(q, 0.0) - jnp.log1p(jnp.exp(-jnp.abs(q)))
    out_ref[...] += jnp.full((1, 1), -(jnp.sum(lsp) + jnp.sum(lsq)),
                             jnp.float32)


@jax.jit
def kernel(target_word, context_word, negative_example, W_target, W_context):
    neg_flat = negative_example.reshape(BATCH * NNEG)
    sc = _make_sc_call()
    embt, embc, negacc = sc(target_word.astype(jnp.int32),
                            context_word.astype(jnp.int32),
                            neg_flat.astype(jnp.int32),
                            W_target, W_context)
    nsteps = BATCH // ROWS_PER_STEP
    loss = pl.pallas_call(
        _loss_kernel,
        grid=(nsteps,),
        in_specs=[
            pl.BlockSpec((ROWS_PER_STEP, DIM), lambda i: (i, 0)),
            pl.BlockSpec((ROWS_PER_STEP, DIM), lambda i: (i, 0)),
            pl.BlockSpec((ROWS_PER_STEP, DIM), lambda i: (i, 0)),
        ],
        out_specs=pl.BlockSpec((1, 1), lambda i: (0, 0)),
        out_shape=jax.ShapeDtypeStruct((1, 1), jnp.float32),
    )(embt, embc, negacc)
    return loss[0, 0]


# speedup vs baseline: 9.2106x; 1.8139x over previous
import jax
import jax.numpy as jnp
from jax import lax
from jax.experimental import pallas as pl
from jax.experimental.pallas import tpu as pltpu
from jax.experimental.pallas import tpu_sc as plsc

VOCAB = 1000000
DIM = 64
BATCH = 16384
NNEG = 20

C = 4096
CB = 2 * C
NBLK = (VOCAB + CB - 1) // CB
VR = NBLK * C
TROWS = 2 * VR

NC = 2
NS = 16
L = 16
NW = NC * NS
BPW = BATCH // NW
CHUNK = 64
NCHUNK = BPW // CHUNK
NIDX = CHUNK * NNEG


def _tr_kernel(x_ref, o_ref):
    x = x_ref[...]
    o_ref[:, 0:DIM] = jnp.transpose(x[:, 0:C], (1, 0))
    o_ref[:, DIM:128] = jnp.transpose(x[:, C:CB], (1, 0))


def _relayout_table(w):
    w128 = pl.pallas_call(
        _tr_kernel,
        grid=(NBLK,),
        in_specs=[pl.BlockSpec((DIM, CB), lambda i: (0, i))],
        out_specs=pl.BlockSpec((C, 128), lambda i: (i, 0)),
        out_shape=jax.ShapeDtypeStruct((VR, 128), jnp.float32),
    )(w.T)
    return w128.reshape(TROWS, DIM)


def _transform_idx(ref, n):
    for j in range(n // L):
        v = ref[pl.ds(j * L, L)]
        g = ((v >> 13) << 13) | ((v & (C - 1)) << 1) | ((v >> 12) & 1)
        ref[pl.ds(j * L, L)] = g


def _sc_kernel_body(tgt_hbm, ctx_hbm, neg_hbm, wt_hbm, wc_hbm,
                    embt_hbm, embc_hbm, negacc_hbm,
                    tgt_idx_v, ctx_idx_v, neg_idx_v,
                    trow_v, crow_v, negrows_v, negacc_v, sem):
    wid = lax.axis_index("s") * NC + lax.axis_index("c")
    wbase = wid * BPW

    def chunk_body(ch, _):
        base = wbase + ch * CHUNK
        pltpu.sync_copy(tgt_hbm.at[pl.ds(base, CHUNK)], tgt_idx_v)
        pltpu.sync_copy(ctx_hbm.at[pl.ds(base, CHUNK)], ctx_idx_v)
        pltpu.sync_copy(neg_hbm.at[pl.ds(base * NNEG, NIDX)], neg_idx_v)
        _transform_idx(tgt_idx_v, CHUNK)
        _transform_idx(ctx_idx_v, CHUNK)
        _transform_idx(neg_idx_v, NIDX)
        cps = [
            pltpu.async_copy(wt_hbm.at[tgt_idx_v], trow_v, sem),
            pltpu.async_copy(wc_hbm.at[ctx_idx_v], crow_v, sem),
        ]
        for j in range(NIDX // 128):
            cps.append(
                pltpu.async_copy(wc_hbm.at[neg_idx_v.at[pl.ds(j * 128, 128)]],
                                 negrows_v.at[pl.ds(j * 128, 128)], sem))
        for cp in cps:
            cp.wait()

        def row_body(r, _):
            nbase = r * NNEG
            for k in range(DIM // L):
                acc = negrows_v[nbase, pl.ds(k * L, L)]
                for n in range(1, NNEG):
                    acc = acc + negrows_v[nbase + n, pl.ds(k * L, L)]
                negacc_v[r, pl.ds(k * L, L)] = acc
            return 0

        lax.fori_loop(0, CHUNK, row_body, 0)

        pltpu.sync_copy(trow_v, embt_hbm.at[pl.ds(base, CHUNK)])
        pltpu.sync_copy(crow_v, embc_hbm.at[pl.ds(base, CHUNK)])
        pltpu.sync_copy(negacc_v, negacc_hbm.at[pl.ds(base, CHUNK)])
        return 0

    lax.fori_loop(0, NCHUNK, chunk_body, 0)


def _make_sc_call():
    mesh = plsc.VectorSubcoreMesh(core_axis_name="c", subcore_axis_name="s",
                                  num_cores=NC, num_subcores=NS)
    return pl.kernel(
        _sc_kernel_body,
        out_type=(
            jax.ShapeDtypeStruct((BATCH, DIM), jnp.float32),
            jax.ShapeDtypeStruct((BATCH, DIM), jnp.float32),
            jax.ShapeDtypeStruct((BATCH, DIM), jnp.float32),
        ),
        mesh=mesh,
        compiler_params=pltpu.CompilerParams(use_tc_tiling_on_sc=False),
        scratch_types=[
            pltpu.VMEM((CHUNK,), jnp.int32),
            pltpu.VMEM((CHUNK,), jnp.int32),
            pltpu.VMEM((NIDX,), jnp.int32),
            pltpu.VMEM((CHUNK, DIM), jnp.float32),
            pltpu.VMEM((CHUNK, DIM), jnp.float32),
            pltpu.VMEM((NIDX, DIM), jnp.float32),
            pltpu.VMEM((CHUNK, DIM), jnp.float32),
            pltpu.SemaphoreType.DMA,
        ],
    )


ROWS_PER_STEP = 2048


def _loss_kernel(t_ref, c_ref, n_ref, out_ref):
    i = pl.program_id(0)

    @pl.when(i == 0)
    def _():
        out_ref[...] = jnp.zeros_like(out_ref)

    t = t_ref[...]
    p = jnp.sum(t * c_ref[...], axis=1)
    q = -jnp.sum(t * n_ref[...], axis=1)
    lsp = jnp.minimum(p, 0.0) - jnp.log1p(jnp.exp(-jnp.abs(p)))
    lsq = jnp.minimum(q, 0.0) - jnp.log1p(jnp.exp(-jnp.abs(q)))
    out_ref[...] += jnp.full((1, 1), -(jnp.sum(lsp) + jnp.sum(lsq)),
                             jnp.float32)


@jax.jit
def kernel(target_word, context_word, negative_example, W_target, W_context):
    neg_flat = negative_example.reshape(BATCH * NNEG)
    wt64 = _relayout_table(W_target)
    wc64 = _relayout_table(W_context)
    sc = _make_sc_call()
    embt, embc, negacc = sc(target_word.astype(jnp.int32),
                            context_word.astype(jnp.int32),
                            neg_flat.astype(jnp.int32),
                            wt64, wc64)
    nsteps = BATCH // ROWS_PER_STEP
    loss = pl.pallas_call(
        _loss_kernel,
        grid=(nsteps,),
        in_specs=[
            pl.BlockSpec((ROWS_PER_STEP, DIM), lambda i: (i, 0)),
            pl.BlockSpec((ROWS_PER_STEP, DIM), lambda i: (i, 0)),
            pl.BlockSpec((ROWS_PER_STEP, DIM), lambda i: (i, 0)),
        ],
        out_specs=pl.BlockSpec((1, 1), lambda i: (0, 0)),
        out_shape=jax.ShapeDtypeStruct((1, 1), jnp.float32),
    )(embt, embc, negacc)
    return loss[0, 0]


# speedup vs baseline: 10.4580x; 1.1354x over previous
import jax
import jax.numpy as jnp
from jax import lax
from jax.experimental import pallas as pl
from jax.experimental.pallas import tpu as pltpu
from jax.experimental.pallas import tpu_sc as plsc

VOCAB = 1000000
DIM = 64
BATCH = 16384
NNEG = 20

C = 4096
CB = 2 * C
NBLK = (VOCAB + CB - 1) // CB
VR = NBLK * C
TROWS = 2 * VR

NC = 2
NS = 16
L = 16
NW = NC * NS
BPW = BATCH // NW
CHUNK = 32
NCHUNK = BPW // CHUNK
NIDX = CHUNK * NNEG


def _tr_kernel(x_ref, i_ref, o_ref):
    x = x_ref[...]
    ident = i_ref[...]
    dn = (((0,), (0,)), ((), ()))
    o_ref[:, 0:DIM] = lax.dot_general(x[:, 0:C], ident, dn,
                                      preferred_element_type=jnp.float32)
    o_ref[:, DIM:128] = lax.dot_general(x[:, C:CB], ident, dn,
                                        preferred_element_type=jnp.float32)


def _relayout_table(w):
    ident = jnp.eye(DIM, dtype=jnp.float32)
    w128 = pl.pallas_call(
        _tr_kernel,
        grid=(NBLK,),
        in_specs=[pl.BlockSpec((DIM, CB), lambda i: (0, i)),
                  pl.BlockSpec((DIM, DIM), lambda i: (0, 0))],
        out_specs=pl.BlockSpec((C, 128), lambda i: (i, 0)),
        out_shape=jax.ShapeDtypeStruct((VR, 128), jnp.float32),
    )(w.T, ident)
    return w128.reshape(TROWS, DIM)


def _transform_ref(ref, n):
    def body(j, _):
        v = ref[pl.ds(j * L, L)]
        g = ((v >> 13) << 13) | ((v & (C - 1)) << 1) | ((v >> 12) & 1)
        ref[pl.ds(j * L, L)] = g
        return 0
    lax.fori_loop(0, n // L, body, 0)


def _sc_kernel_body(tgt_hbm, ctx_hbm, neg_hbm, wt_hbm, wc_hbm,
                    pos_hbm, negsum_hbm,
                    tgt_idx_v, ctx_idx_v, neg_idx_v,
                    trow_a, crow_a, negrows_a,
                    trow_b, crow_b, negrows_b,
                    pos_out_v, negsum_out_v, fold_v, out16p_v, out16n_v,
                    sem_a, sem_b):
    wid = lax.axis_index("s") * NC + lax.axis_index("c")
    wbase = wid * BPW

    pltpu.sync_copy(tgt_hbm.at[pl.ds(wbase, BPW)], tgt_idx_v)
    pltpu.sync_copy(ctx_hbm.at[pl.ds(wbase, BPW)], ctx_idx_v)
    pltpu.sync_copy(neg_hbm.at[pl.ds(wbase * NNEG, BPW * NNEG)], neg_idx_v)
    _transform_ref(tgt_idx_v, BPW)
    _transform_ref(ctx_idx_v, BPW)
    _transform_ref(neg_idx_v, BPW * NNEG)

    def copies(ch, trow, crow, negrows, sem):
        cps = [
            pltpu.make_async_copy(
                wt_hbm.at[tgt_idx_v.at[pl.ds(ch * CHUNK, CHUNK)]], trow, sem),
            pltpu.make_async_copy(
                wc_hbm.at[ctx_idx_v.at[pl.ds(ch * CHUNK, CHUNK)]], crow, sem),
        ]
        for j in range(NIDX // 128):
            cps.append(pltpu.make_async_copy(
                wc_hbm.at[neg_idx_v.at[pl.ds(ch * NIDX + j * 128, 128)]],
                negrows.at[pl.ds(j * 128, 128)], sem))
        return cps

    def issue(ch, trow, crow, negrows, sem):
        for cp in copies(ch, trow, crow, negrows, sem):
            cp.start()

    def wait(ch, trow, crow, negrows, sem):
        for cp in copies(ch, trow, crow, negrows, sem):
            cp.wait()

    def compute(ch, trow, crow, negrows):
        for g2 in range(CHUNK // L):
            def row_body(r16, _):
                r = g2 * L + r16
                t = [trow[r, pl.ds(k * L, L)] for k in range(DIM // L)]
                c = [crow[r, pl.ds(k * L, L)] for k in range(DIM // L)]
                pv = t[0] * c[0]
                for k in range(1, DIM // L):
                    pv = pv + t[k] * c[k]
                nbase = r * NNEG
                acc = [negrows[nbase, pl.ds(k * L, L)]
                       for k in range(DIM // L)]
                for n in range(1, NNEG):
                    for k in range(DIM // L):
                        acc[k] = acc[k] + negrows[nbase + n, pl.ds(k * L, L)]
                nv = acc[0] * t[0]
                for k in range(1, DIM // L):
                    nv = nv + acc[k] * t[k]
                for d in (8, 4, 2, 1):
                    fold_v[pl.ds(0, L)] = pv
                    pv = pv + fold_v[pl.ds(d, L)]
                for d in (8, 4, 2, 1):
                    fold_v[pl.ds(0, L)] = nv
                    nv = nv + fold_v[pl.ds(d, L)]
                out16p_v[pl.ds(r16, L)] = pv
                out16n_v[pl.ds(r16, L)] = nv
                return 0

            lax.fori_loop(0, L, row_body, 0)
            off = ch * CHUNK + g2 * L
            pos_out_v[pl.ds(off, L)] = out16p_v[pl.ds(0, L)]
            negsum_out_v[pl.ds(off, L)] = out16n_v[pl.ds(0, L)]

    issue(0, trow_a, crow_a, negrows_a, sem_a)
    issue(1, trow_b, crow_b, negrows_b, sem_b)

    def pair_body(g, _):
        ch_a = 2 * g
        wait(ch_a, trow_a, crow_a, negrows_a, sem_a)
        compute(ch_a, trow_a, crow_a, negrows_a)
        issue(ch_a + 2, trow_a, crow_a, negrows_a, sem_a)
        ch_b = 2 * g + 1
        wait(ch_b, trow_b, crow_b, negrows_b, sem_b)
        compute(ch_b, trow_b, crow_b, negrows_b)
        issue(ch_b + 2, trow_b, crow_b, negrows_b, sem_b)
        return 0

    lax.fori_loop(0, NCHUNK // 2 - 1, pair_body, 0)
    last_a = NCHUNK - 2
    wait(last_a, trow_a, crow_a, negrows_a, sem_a)
    compute(last_a, trow_a, crow_a, negrows_a)
    last_b = NCHUNK - 1
    wait(last_b, trow_b, crow_b, negrows_b, sem_b)
    compute(last_b, trow_b, crow_b, negrows_b)

    pltpu.sync_copy(pos_out_v, pos_hbm.at[pl.ds(wbase, BPW)])
    pltpu.sync_copy(negsum_out_v, negsum_hbm.at[pl.ds(wbase, BPW)])


def _make_sc_call():
    mesh = plsc.VectorSubcoreMesh(core_axis_name="c", subcore_axis_name="s",
                                  num_cores=NC, num_subcores=NS)
    return pl.kernel(
        _sc_kernel_body,
        out_type=(
            jax.ShapeDtypeStruct((BATCH,), jnp.float32),
            jax.ShapeDtypeStruct((BATCH,), jnp.float32),
        ),
        mesh=mesh,
        compiler_params=pltpu.CompilerParams(use_tc_tiling_on_sc=False),
        scratch_types=[
            pltpu.VMEM((BPW,), jnp.int32),
            pltpu.VMEM((BPW,), jnp.int32),
            pltpu.VMEM((BPW * NNEG,), jnp.int32),
            pltpu.VMEM((CHUNK, DIM), jnp.float32),
            pltpu.VMEM((CHUNK, DIM), jnp.float32),
            pltpu.VMEM((NIDX, DIM), jnp.float32),
            pltpu.VMEM((CHUNK, DIM), jnp.float32),
            pltpu.VMEM((CHUNK, DIM), jnp.float32),
            pltpu.VMEM((NIDX, DIM), jnp.float32),
            pltpu.VMEM((BPW,), jnp.float32),
            pltpu.VMEM((BPW,), jnp.float32),
            pltpu.VMEM((32,), jnp.float32),
            pltpu.VMEM((32,), jnp.float32),
            pltpu.VMEM((32,), jnp.float32),
            pltpu.SemaphoreType.DMA,
            pltpu.SemaphoreType.DMA,
        ],
    )


def _loss_kernel(p_ref, n_ref, out_ref):
    p = p_ref[...]
    q = -n_ref[...]
    lsp = jnp.minimum(p, 0.0) - jnp.log1p(jnp.exp(-jnp.abs(p)))
    lsq = jnp.minimum(q, 0.0) - jnp.log1p(jnp.exp(-jnp.abs(q)))
    out_ref[...] = jnp.full((1, 1), -(jnp.sum(lsp) + jnp.sum(lsq)),
                            jnp.float32)


@jax.jit
def kernel(target_word, context_word, negative_example, W_target, W_context):
    neg_flat = negative_example.reshape(BATCH * NNEG)
    wt64 = _relayout_table(W_target)
    wc64 = _relayout_table(W_context)
    sc = _make_sc_call()
    pos, negsum = sc(target_word.astype(jnp.int32),
                     context_word.astype(jnp.int32),
                     neg_flat.astype(jnp.int32),
                     wt64, wc64)
    loss = pl.pallas_call(
        _loss_kernel,
        out_shape=jax.ShapeDtypeStruct((1, 1), jnp.float32),
    )(pos.reshape(128, 128), negsum.reshape(128, 128))
    return loss[0, 0]


# speedup vs baseline: 11.6980x; 1.1186x over previous
import jax
import jax.numpy as jnp
from jax import lax
from jax.experimental import pallas as pl
from jax.experimental.pallas import tpu as pltpu
from jax.experimental.pallas import tpu_sc as plsc

VOCAB = 1000000
DIM = 64
BATCH = 16384
NNEG = 20

LOG2C = 13
C = 1 << LOG2C
CB = 2 * C
NBLK = (VOCAB + CB - 1) // CB
VR = NBLK * C
TROWS = 2 * VR

NC = 2
NS = 16
L = 16
NW = NC * NS
BPW = BATCH // NW
CHUNK = 32
NCHUNK = BPW // CHUNK
NIDX = CHUNK * NNEG


def _tr_kernel(x_ref, i_ref, o_ref):
    x = x_ref[...]
    ident = i_ref[...]
    dn = (((0,), (0,)), ((), ()))
    o_ref[:, 0:DIM] = lax.dot_general(x[:, 0:C], ident, dn,
                                      preferred_element_type=jnp.float32)
    o_ref[:, DIM:128] = lax.dot_general(x[:, C:CB], ident, dn,
                                        preferred_element_type=jnp.float32)


def _relayout_table(w):
    ident = jnp.eye(DIM, dtype=jnp.float32)
    w128 = pl.pallas_call(
        _tr_kernel,
        grid=(NBLK,),
        in_specs=[pl.BlockSpec((DIM, CB), lambda i: (0, i)),
                  pl.BlockSpec((DIM, DIM), lambda i: (0, 0))],
        out_specs=pl.BlockSpec((C, 128), lambda i: (i, 0)),
        out_shape=jax.ShapeDtypeStruct((VR, 128), jnp.float32),
    )(w.T, ident)
    return w128.reshape(TROWS, DIM)


def _transform_ref(ref, n):
    def body(j, _):
        v = ref[pl.ds(j * L, L)]
        g = (((v >> (LOG2C + 1)) << (LOG2C + 1))
             | ((v & (C - 1)) << 1) | ((v >> LOG2C) & 1))
        ref[pl.ds(j * L, L)] = g
        return 0
    lax.fori_loop(0, n // L, body, 0)


def _sc_kernel_body(tgt_hbm, ctx_hbm, neg_hbm, wt_hbm, wc_hbm,
                    pos_hbm, negsum_hbm,
                    tgt_idx_v, ctx_idx_v, neg_idx_v,
                    trow_a, crow_a, negrows_a,
                    trow_b, crow_b, negrows_b,
                    pos_out_v, negsum_out_v, fold_v, out16p_v, out16n_v,
                    sem_a, sem_b):
    wid = lax.axis_index("s") * NC + lax.axis_index("c")
    wbase = wid * BPW

    pltpu.sync_copy(tgt_hbm.at[pl.ds(wbase, BPW)], tgt_idx_v)
    pltpu.sync_copy(ctx_hbm.at[pl.ds(wbase, BPW)], ctx_idx_v)
    pltpu.sync_copy(neg_hbm.at[pl.ds(wbase * NNEG, BPW * NNEG)], neg_idx_v)
    _transform_ref(tgt_idx_v, BPW)
    _transform_ref(ctx_idx_v, BPW)
    _transform_ref(neg_idx_v, BPW * NNEG)

    def copies(ch, trow, crow, negrows, sem):
        cps = [
            pltpu.make_async_copy(
                wt_hbm.at[tgt_idx_v.at[pl.ds(ch * CHUNK, CHUNK)]], trow, sem),
            pltpu.make_async_copy(
                wc_hbm.at[ctx_idx_v.at[pl.ds(ch * CHUNK, CHUNK)]], crow, sem),
        ]
        for j in range(NIDX // 128):
            cps.append(pltpu.make_async_copy(
                wc_hbm.at[neg_idx_v.at[pl.ds(ch * NIDX + j * 128, 128)]],
                negrows.at[pl.ds(j * 128, 128)], sem))
        return cps

    def issue(ch, trow, crow, negrows, sem):
        for cp in copies(ch, trow, crow, negrows, sem):
            cp.start()

    def wait(ch, trow, crow, negrows, sem):
        for cp in copies(ch, trow, crow, negrows, sem):
            cp.wait()

    def compute(ch, trow, crow, negrows):
        for g2 in range(CHUNK // L):
            def row_body(r16, _):
                r = g2 * L + r16
                t = [trow[r, pl.ds(k * L, L)] for k in range(DIM // L)]
                c = [crow[r, pl.ds(k * L, L)] for k in range(DIM // L)]
                pv = t[0] * c[0]
                for k in range(1, DIM // L):
                    pv = pv + t[k] * c[k]
                nbase = r * NNEG
                acc = [negrows[nbase, pl.ds(k * L, L)]
                       for k in range(DIM // L)]
                for n in range(1, NNEG):
                    for k in range(DIM // L):
                        acc[k] = acc[k] + negrows[nbase + n, pl.ds(k * L, L)]
                nv = acc[0] * t[0]
                for k in range(1, DIM // L):
                    nv = nv + acc[k] * t[k]
                for d in (8, 4, 2, 1):
                    fold_v[pl.ds(0, L)] = pv
                    pv = pv + fold_v[pl.ds(d, L)]
                for d in (8, 4, 2, 1):
                    fold_v[pl.ds(0, L)] = nv
                    nv = nv + fold_v[pl.ds(d, L)]
                out16p_v[pl.ds(r16, L)] = pv
                out16n_v[pl.ds(r16, L)] = nv
                return 0

            lax.fori_loop(0, L, row_body, 0)
            off = ch * CHUNK + g2 * L
            pos_out_v[pl.ds(off, L)] = out16p_v[pl.ds(0, L)]
            negsum_out_v[pl.ds(off, L)] = out16n_v[pl.ds(0, L)]

    issue(0, trow_a, crow_a, negrows_a, sem_a)
    issue(1, trow_b, crow_b, negrows_b, sem_b)

    def pair_body(g, _):
        ch_a = 2 * g
        wait(ch_a, trow_a, crow_a, negrows_a, sem_a)
        compute(ch_a, trow_a, crow_a, negrows_a)
        issue(ch_a + 2, trow_a, crow_a, negrows_a, sem_a)
        ch_b = 2 * g + 1
        wait(ch_b, trow_b, crow_b, negrows_b, sem_b)
        compute(ch_b, trow_b, crow_b, negrows_b)
        issue(ch_b + 2, trow_b, crow_b, negrows_b, sem_b)
        return 0

    lax.fori_loop(0, NCHUNK // 2 - 1, pair_body, 0)
    last_a = NCHUNK - 2
    wait(last_a, trow_a, crow_a, negrows_a, sem_a)
    compute(last_a, trow_a, crow_a, negrows_a)
    last_b = NCHUNK - 1
    wait(last_b, trow_b, crow_b, negrows_b, sem_b)
    compute(last_b, trow_b, crow_b, negrows_b)

    pltpu.sync_copy(pos_out_v, pos_hbm.at[pl.ds(wbase, BPW)])
    pltpu.sync_copy(negsum_out_v, negsum_hbm.at[pl.ds(wbase, BPW)])


def _make_sc_call():
    mesh = plsc.VectorSubcoreMesh(core_axis_name="c", subcore_axis_name="s",
                                  num_cores=NC, num_subcores=NS)
    return pl.kernel(
        _sc_kernel_body,
        out_type=(
            jax.ShapeDtypeStruct((BATCH,), jnp.float32),
            jax.ShapeDtypeStruct((BATCH,), jnp.float32),
        ),
        mesh=mesh,
        compiler_params=pltpu.CompilerParams(use_tc_tiling_on_sc=False),
        scratch_types=[
            pltpu.VMEM((BPW,), jnp.int32),
            pltpu.VMEM((BPW,), jnp.int32),
            pltpu.VMEM((BPW * NNEG,), jnp.int32),
            pltpu.VMEM((CHUNK, DIM), jnp.float32),
            pltpu.VMEM((CHUNK, DIM), jnp.float32),
            pltpu.VMEM((NIDX, DIM), jnp.float32),
            pltpu.VMEM((CHUNK, DIM), jnp.float32),
            pltpu.VMEM((CHUNK, DIM), jnp.float32),
            pltpu.VMEM((NIDX, DIM), jnp.float32),
            pltpu.VMEM((BPW,), jnp.float32),
            pltpu.VMEM((BPW,), jnp.float32),
            pltpu.VMEM((32,), jnp.float32),
            pltpu.VMEM((32,), jnp.float32),
            pltpu.VMEM((32,), jnp.float32),
            pltpu.SemaphoreType.DMA,
            pltpu.SemaphoreType.DMA,
        ],
    )


def _loss_kernel(p_ref, n_ref, out_ref):
    p = p_ref[...]
    q = -n_ref[...]
    lsp = jnp.minimum(p, 0.0) - jnp.log1p(jnp.exp(-jnp.abs(p)))
    lsq = jnp.minimum(q, 0.0) - jnp.log1p(jnp.exp(-jnp.abs(q)))
    out_ref[...] = jnp.full((1, 1), -(jnp.sum(lsp) + jnp.sum(lsq)),
                            jnp.float32)


@jax.jit
def kernel(target_word, context_word, negative_example, W_target, W_context):
    neg_flat = negative_example.reshape(BATCH * NNEG)
    wt64 = _relayout_table(W_target)
    wc64 = _relayout_table(W_context)
    sc = _make_sc_call()
    pos, negsum = sc(target_word.astype(jnp.int32),
                     context_word.astype(jnp.int32),
                     neg_flat.astype(jnp.int32),
                     wt64, wc64)
    loss = pl.pallas_call(
        _loss_kernel,
        out_shape=jax.ShapeDtypeStruct((1, 1), jnp.float32),
    )(pos.reshape(128, 128), negsum.reshape(128, 128))
    return loss[0, 0]


# speedup vs baseline: 15.0363x; 1.2854x over previous
import jax
import jax.numpy as jnp
from jax import lax
from jax.experimental import pallas as pl
from jax.experimental.pallas import tpu as pltpu
from jax.experimental.pallas import tpu_sc as plsc

VOCAB = 1000000
DIM = 64
BATCH = 16384
NNEG = 20

LOG2C = 13
C = 1 << LOG2C
CB = 2 * C
NBLK = (VOCAB + CB - 1) // CB
VR = NBLK * C
TROWS = 2 * VR

NC = 2
NS = 16
L = 16
NW = NC * NS
BPW = BATCH // NW
CHUNK = 32
NCHUNK = BPW // CHUNK
NIDX = CHUNK * NNEG


def _tr_kernel(x_ref, i_ref, o_ref):
    x = x_ref[...]
    ident = i_ref[...]
    dn = (((0,), (0,)), ((), ()))
    xcat = jnp.concatenate([x[:, 0:C], x[:, C:CB]], axis=0)
    o_ref[...] = lax.dot_general(xcat, ident, dn,
                                 preferred_element_type=jnp.float32)


def _relayout_table(w):
    ident = jnp.eye(2 * DIM, dtype=jnp.float32)
    w128 = pl.pallas_call(
        _tr_kernel,
        grid=(NBLK,),
        in_specs=[pl.BlockSpec((DIM, CB), lambda i: (0, i)),
                  pl.BlockSpec((2 * DIM, 128), lambda i: (0, 0))],
        out_specs=pl.BlockSpec((C, 128), lambda i: (i, 0)),
        out_shape=jax.ShapeDtypeStruct((VR, 128), jnp.float32),
    )(w.T, ident)
    return w128.reshape(TROWS, DIM)


def _transform_ref(ref, n):
    def body(j, _):
        v = ref[pl.ds(j * L, L)]
        g = (((v >> (LOG2C + 1)) << (LOG2C + 1))
             | ((v & (C - 1)) << 1) | ((v >> LOG2C) & 1))
        ref[pl.ds(j * L, L)] = g
        return 0
    lax.fori_loop(0, n // L, body, 0)


def _sc_kernel_body(tgt_hbm, ctx_hbm, neg_hbm, wt_hbm, wc_hbm,
                    pos_hbm, negsum_hbm,
                    tgt_idx_v, ctx_idx_v, neg_idx_v,
                    trow_a, crow_a, negrows_a,
                    trow_b, crow_b, negrows_b,
                    pos_out_v, negsum_out_v, fold_v, out16p_v, out16n_v,
                    sem_a, sem_b):
    wid = lax.axis_index("s") * NC + lax.axis_index("c")
    wbase = wid * BPW

    pltpu.sync_copy(tgt_hbm.at[pl.ds(wbase, BPW)], tgt_idx_v)
    pltpu.sync_copy(ctx_hbm.at[pl.ds(wbase, BPW)], ctx_idx_v)
    pltpu.sync_copy(neg_hbm.at[pl.ds(wbase * NNEG, BPW * NNEG)], neg_idx_v)
    _transform_ref(tgt_idx_v, BPW)
    _transform_ref(ctx_idx_v, BPW)
    _transform_ref(neg_idx_v, BPW * NNEG)

    def copies(ch, trow, crow, negrows, sem):
        cps = [
            pltpu.make_async_copy(
                wt_hbm.at[tgt_idx_v.at[pl.ds(ch * CHUNK, CHUNK)]], trow, sem),
            pltpu.make_async_copy(
                wc_hbm.at[ctx_idx_v.at[pl.ds(ch * CHUNK, CHUNK)]], crow, sem),
        ]
        for j in range(NIDX // 128):
            cps.append(pltpu.make_async_copy(
                wc_hbm.at[neg_idx_v.at[pl.ds(ch * NIDX + j * 128, 128)]],
                negrows.at[pl.ds(j * 128, 128)], sem))
        return cps

    def issue(ch, trow, crow, negrows, sem):
        for cp in copies(ch, trow, crow, negrows, sem):
            cp.start()

    def wait(ch, trow, crow, negrows, sem):
        for cp in copies(ch, trow, crow, negrows, sem):
            cp.wait()

    def compute(ch, trow, crow, negrows):
        for g2 in range(CHUNK // L):
            def row_body(r16, _):
                r = g2 * L + r16
                t = [trow[r, pl.ds(k * L, L)] for k in range(DIM // L)]
                c = [crow[r, pl.ds(k * L, L)] for k in range(DIM // L)]
                pv = t[0] * c[0]
                for k in range(1, DIM // L):
                    pv = pv + t[k] * c[k]
                nbase = r * NNEG
                acc = [negrows[nbase, pl.ds(k * L, L)]
                       for k in range(DIM // L)]
                for n in range(1, NNEG):
                    for k in range(DIM // L):
                        acc[k] = acc[k] + negrows[nbase + n, pl.ds(k * L, L)]
                nv = acc[0] * t[0]
                for k in range(1, DIM // L):
                    nv = nv + acc[k] * t[k]
                for d in (8, 4, 2, 1):
                    fold_v[pl.ds(0, L)] = pv
                    pv = pv + fold_v[pl.ds(d, L)]
                for d in (8, 4, 2, 1):
                    fold_v[pl.ds(0, L)] = nv
                    nv = nv + fold_v[pl.ds(d, L)]
                out16p_v[pl.ds(r16, L)] = pv
                out16n_v[pl.ds(r16, L)] = nv
                return 0

            lax.fori_loop(0, L, row_body, 0)
            off = ch * CHUNK + g2 * L
            pos_out_v[pl.ds(off, L)] = out16p_v[pl.ds(0, L)]
            negsum_out_v[pl.ds(off, L)] = out16n_v[pl.ds(0, L)]

    issue(0, trow_a, crow_a, negrows_a, sem_a)
    issue(1, trow_b, crow_b, negrows_b, sem_b)

    def pair_body(g, _):
        ch_a = 2 * g
        wait(ch_a, trow_a, crow_a, negrows_a, sem_a)
        compute(ch_a, trow_a, crow_a, negrows_a)
        issue(ch_a + 2, trow_a, crow_a, negrows_a, sem_a)
        ch_b = 2 * g + 1
        wait(ch_b, trow_b, crow_b, negrows_b, sem_b)
        compute(ch_b, trow_b, crow_b, negrows_b)
        issue(ch_b + 2, trow_b, crow_b, negrows_b, sem_b)
        return 0

    lax.fori_loop(0, NCHUNK // 2 - 1, pair_body, 0)
    last_a = NCHUNK - 2
    wait(last_a, trow_a, crow_a, negrows_a, sem_a)
    compute(last_a, trow_a, crow_a, negrows_a)
    last_b = NCHUNK - 1
    wait(last_b, trow_b, crow_b, negrows_b, sem_b)
    compute(last_b, trow_b, crow_b, negrows_b)

    pltpu.sync_copy(pos_out_v, pos_hbm.at[pl.ds(wbase, BPW)])
    pltpu.sync_copy(negsum_out_v, negsum_hbm.at[pl.ds(wbase, BPW)])


def _make_sc_call():
    mesh = plsc.VectorSubcoreMesh(core_axis_name="c", subcore_axis_name="s",
                                  num_cores=NC, num_subcores=NS)
    return pl.kernel(
        _sc_kernel_body,
        out_type=(
            jax.ShapeDtypeStruct((BATCH,), jnp.float32),
            jax.ShapeDtypeStruct((BATCH,), jnp.float32),
        ),
        mesh=mesh,
        compiler_params=pltpu.CompilerParams(use_tc_tiling_on_sc=False),
        scratch_types=[
            pltpu.VMEM((BPW,), jnp.int32),
            pltpu.VMEM((BPW,), jnp.int32),
            pltpu.VMEM((BPW * NNEG,), jnp.int32),
            pltpu.VMEM((CHUNK, DIM), jnp.float32),
            pltpu.VMEM((CHUNK, DIM), jnp.float32),
            pltpu.VMEM((NIDX, DIM), jnp.float32),
            pltpu.VMEM((CHUNK, DIM), jnp.float32),
            pltpu.VMEM((CHUNK, DIM), jnp.float32),
            pltpu.VMEM((NIDX, DIM), jnp.float32),
            pltpu.VMEM((BPW,), jnp.float32),
            pltpu.VMEM((BPW,), jnp.float32),
            pltpu.VMEM((32,), jnp.float32),
            pltpu.VMEM((32,), jnp.float32),
            pltpu.VMEM((32,), jnp.float32),
            pltpu.SemaphoreType.DMA,
            pltpu.SemaphoreType.DMA,
        ],
    )


def _loss_kernel(p_ref, n_ref, out_ref):
    p = p_ref[...]
    q = -n_ref[...]
    lsp = jnp.minimum(p, 0.0) - jnp.log1p(jnp.exp(-jnp.abs(p)))
    lsq = jnp.minimum(q, 0.0) - jnp.log1p(jnp.exp(-jnp.abs(q)))
    out_ref[...] = jnp.full((1, 1), -(jnp.sum(lsp) + jnp.sum(lsq)),
                            jnp.float32)


@jax.jit
def kernel(target_word, context_word, negative_example, W_target, W_context):
    neg_flat = negative_example.reshape(BATCH * NNEG)
    wt64 = _relayout_table(W_target)
    wc64 = _relayout_table(W_context)
    sc = _make_sc_call()
    pos, negsum = sc(target_word.astype(jnp.int32),
                     context_word.astype(jnp.int32),
                     neg_flat.astype(jnp.int32),
                     wt64, wc64)
    loss = pl.pallas_call(
        _loss_kernel,
        out_shape=jax.ShapeDtypeStruct((1, 1), jnp.float32),
    )(pos.reshape(128, 128), negsum.reshape(128, 128))
    return loss[0, 0]


# speedup vs baseline: 15.3817x; 1.0230x over previous
import jax
import jax.numpy as jnp
from jax import lax
from jax.experimental import pallas as pl
from jax.experimental.pallas import tpu as pltpu
from jax.experimental.pallas import tpu_sc as plsc

VOCAB = 1000000
DIM = 64
BATCH = 16384
NNEG = 20

LOG2C = 14
C = 1 << LOG2C
CB = 2 * C
NBLK = (VOCAB + CB - 1) // CB
VR = NBLK * C
TROWS = 2 * VR

NC = 2
NS = 16
L = 16
NW = NC * NS
BPW = BATCH // NW
CHUNK = 32
NCHUNK = BPW // CHUNK
NIDX = CHUNK * NNEG


def _tr_kernel(x_ref, i_ref, o_ref):
    x = x_ref[...]
    ident = i_ref[...]
    dn = (((0,), (0,)), ((), ()))
    xcat = jnp.concatenate([x[:, 0:C], x[:, C:CB]], axis=0)
    o_ref[...] = lax.dot_general(xcat, ident, dn,
                                 preferred_element_type=jnp.float32)


def _relayout_table(w):
    ident = jnp.eye(2 * DIM, dtype=jnp.float32)
    w128 = pl.pallas_call(
        _tr_kernel,
        grid=(NBLK,),
        in_specs=[pl.BlockSpec((DIM, CB), lambda i: (0, i)),
                  pl.BlockSpec((2 * DIM, 128), lambda i: (0, 0))],
        out_specs=pl.BlockSpec((C, 128), lambda i: (i, 0)),
        out_shape=jax.ShapeDtypeStruct((VR, 128), jnp.float32),
        compiler_params=pltpu.CompilerParams(
            vmem_limit_bytes=100 * 1024 * 1024),
    )(w.T, ident)
    return w128.reshape(TROWS, DIM)


def _transform_ref(ref, n):
    def body(j, _):
        v = ref[pl.ds(j * L, L)]
        g = (((v >> (LOG2C + 1)) << (LOG2C + 1))
             | ((v & (C - 1)) << 1) | ((v >> LOG2C) & 1))
        ref[pl.ds(j * L, L)] = g
        return 0
    lax.fori_loop(0, n // L, body, 0)


def _sc_kernel_body(tgt_hbm, ctx_hbm, neg_hbm, wt_hbm, wc_hbm,
                    pos_hbm, negsum_hbm,
                    tgt_idx_v, ctx_idx_v, neg_idx_v,
                    trow_a, crow_a, negrows_a,
                    trow_b, crow_b, negrows_b,
                    pos_out_v, negsum_out_v, fold_v, out16p_v, out16n_v,
                    sem_a, sem_b):
    wid = lax.axis_index("s") * NC + lax.axis_index("c")
    wbase = wid * BPW

    pltpu.sync_copy(tgt_hbm.at[pl.ds(wbase, BPW)], tgt_idx_v)
    pltpu.sync_copy(ctx_hbm.at[pl.ds(wbase, BPW)], ctx_idx_v)
    pltpu.sync_copy(neg_hbm.at[pl.ds(wbase * NNEG, BPW * NNEG)], neg_idx_v)
    _transform_ref(tgt_idx_v, BPW)
    _transform_ref(ctx_idx_v, BPW)
    _transform_ref(neg_idx_v, BPW * NNEG)

    def copies(ch, trow, crow, negrows, sem):
        cps = [
            pltpu.make_async_copy(
                wt_hbm.at[tgt_idx_v.at[pl.ds(ch * CHUNK, CHUNK)]], trow, sem),
            pltpu.make_async_copy(
                wc_hbm.at[ctx_idx_v.at[pl.ds(ch * CHUNK, CHUNK)]], crow, sem),
        ]
        for j in range(NIDX // 128):
            cps.append(pltpu.make_async_copy(
                wc_hbm.at[neg_idx_v.at[pl.ds(ch * NIDX + j * 128, 128)]],
                negrows.at[pl.ds(j * 128, 128)], sem))
        return cps

    def issue(ch, trow, crow, negrows, sem):
        for cp in copies(ch, trow, crow, negrows, sem):
            cp.start()

    def wait(ch, trow, crow, negrows, sem):
        for cp in copies(ch, trow, crow, negrows, sem):
            cp.wait()

    def compute(ch, trow, crow, negrows):
        for g2 in range(CHUNK // L):
            def row_body(r16, _):
                r = g2 * L + r16
                t = [trow[r, pl.ds(k * L, L)] for k in range(DIM // L)]
                c = [crow[r, pl.ds(k * L, L)] for k in range(DIM // L)]
                pv = t[0] * c[0]
                for k in range(1, DIM // L):
                    pv = pv + t[k] * c[k]
                nbase = r * NNEG
                acc = [negrows[nbase, pl.ds(k * L, L)]
                       for k in range(DIM // L)]
                for n in range(1, NNEG):
                    for k in range(DIM // L):
                        acc[k] = acc[k] + negrows[nbase + n, pl.ds(k * L, L)]
                nv = acc[0] * t[0]
                for k in range(1, DIM // L):
                    nv = nv + acc[k] * t[k]
                for d in (8, 4, 2, 1):
                    fold_v[pl.ds(0, L)] = pv
                    pv = pv + fold_v[pl.ds(d, L)]
                for d in (8, 4, 2, 1):
                    fold_v[pl.ds(0, L)] = nv
                    nv = nv + fold_v[pl.ds(d, L)]
                out16p_v[pl.ds(r16, L)] = pv
                out16n_v[pl.ds(r16, L)] = nv
                return 0

            lax.fori_loop(0, L, row_body, 0)
            off = ch * CHUNK + g2 * L
            pos_out_v[pl.ds(off, L)] = out16p_v[pl.ds(0, L)]
            negsum_out_v[pl.ds(off, L)] = out16n_v[pl.ds(0, L)]

    issue(0, trow_a, crow_a, negrows_a, sem_a)
    issue(1, trow_b, crow_b, negrows_b, sem_b)

    def pair_body(g, _):
        ch_a = 2 * g
        wait(ch_a, trow_a, crow_a, negrows_a, sem_a)
        compute(ch_a, trow_a, crow_a, negrows_a)
        issue(ch_a + 2, trow_a, crow_a, negrows_a, sem_a)
        ch_b = 2 * g + 1
        wait(ch_b, trow_b, crow_b, negrows_b, sem_b)
        compute(ch_b, trow_b, crow_b, negrows_b)
        issue(ch_b + 2, trow_b, crow_b, negrows_b, sem_b)
        return 0

    lax.fori_loop(0, NCHUNK // 2 - 1, pair_body, 0)
    last_a = NCHUNK - 2
    wait(last_a, trow_a, crow_a, negrows_a, sem_a)
    compute(last_a, trow_a, crow_a, negrows_a)
    last_b = NCHUNK - 1
    wait(last_b, trow_b, crow_b, negrows_b, sem_b)
    compute(last_b, trow_b, crow_b, negrows_b)

    pltpu.sync_copy(pos_out_v, pos_hbm.at[pl.ds(wbase, BPW)])
    pltpu.sync_copy(negsum_out_v, negsum_hbm.at[pl.ds(wbase, BPW)])


def _make_sc_call():
    mesh = plsc.VectorSubcoreMesh(core_axis_name="c", subcore_axis_name="s",
                                  num_cores=NC, num_subcores=NS)
    return pl.kernel(
        _sc_kernel_body,
        out_type=(
            jax.ShapeDtypeStruct((BATCH,), jnp.float32),
            jax.ShapeDtypeStruct((BATCH,), jnp.float32),
        ),
        mesh=mesh,
        compiler_params=pltpu.CompilerParams(use_tc_tiling_on_sc=False),
        scratch_types=[
            pltpu.VMEM((BPW,), jnp.int32),
            pltpu.VMEM((BPW,), jnp.int32),
            pltpu.VMEM((BPW * NNEG,), jnp.int32),
            pltpu.VMEM((CHUNK, DIM), jnp.float32),
            pltpu.VMEM((CHUNK, DIM), jnp.float32),
            pltpu.VMEM((NIDX, DIM), jnp.float32),
            pltpu.VMEM((CHUNK, DIM), jnp.float32),
            pltpu.VMEM((CHUNK, DIM), jnp.float32),
            pltpu.VMEM((NIDX, DIM), jnp.float32),
            pltpu.VMEM((BPW,), jnp.float32),
            pltpu.VMEM((BPW,), jnp.float32),
            pltpu.VMEM((32,), jnp.float32),
            pltpu.VMEM((32,), jnp.float32),
            pltpu.VMEM((32,), jnp.float32),
            pltpu.SemaphoreType.DMA,
            pltpu.SemaphoreType.DMA,
        ],
    )


def _loss_kernel(p_ref, n_ref, out_ref):
    p = p_ref[...]
    q = -n_ref[...]
    lsp = jnp.minimum(p, 0.0) - jnp.log1p(jnp.exp(-jnp.abs(p)))
    lsq = jnp.minimum(q, 0.0) - jnp.log1p(jnp.exp(-jnp.abs(q)))
    out_ref[...] = jnp.full((1, 1), -(jnp.sum(lsp) + jnp.sum(lsq)),
                            jnp.float32)


@jax.jit
def kernel(target_word, context_word, negative_example, W_target, W_context):
    neg_flat = negative_example.reshape(BATCH * NNEG)
    wt64 = _relayout_table(W_target)
    wc64 = _relayout_table(W_context)
    sc = _make_sc_call()
    pos, negsum = sc(target_word.astype(jnp.int32),
                     context_word.astype(jnp.int32),
                     neg_flat.astype(jnp.int32),
                     wt64, wc64)
    loss = pl.pallas_call(
        _loss_kernel,
        out_shape=jax.ShapeDtypeStruct((1, 1), jnp.float32),
    )(pos.reshape(128, 128), negsum.reshape(128, 128))
    return loss[0, 0]
